# baseline (speedup 1.0000x reference)
import functools

import jax
import jax.numpy as jnp
from jax.experimental import pallas as pl
from jax.experimental.pallas import tpu as pltpu

E = 8
K = 2
LB_COEFF = 0.01
BLK = 256


def _routing_body(x_ref, gw_ref, i1_ref, i2_ref, p1_ref, p2_ref, psum_ref):
    x = x_ref[...]
    gw = gw_ref[...]
    scores = jnp.dot(x, gw, preferred_element_type=jnp.float32)
    lane = jax.lax.broadcasted_iota(jnp.int32, scores.shape, 1)
    neg_inf = jnp.float32(-jnp.inf)
    scores = jnp.where(lane < E, scores, neg_inf)
    probs = jax.nn.softmax(scores, axis=-1)
    m1 = jnp.max(probs, axis=1, keepdims=True)
    i1 = jnp.min(jnp.where(probs == m1, lane, 128), axis=1, keepdims=True)
    probs2 = jnp.where(lane == i1, jnp.float32(-1.0), probs)
    m2 = jnp.max(probs2, axis=1, keepdims=True)
    i2 = jnp.min(jnp.where(probs2 == m2, lane, 128), axis=1, keepdims=True)
    e2 = jnp.exp(m2 - m1)
    denom = 1.0 + e2
    p1 = 1.0 / denom
    p2 = e2 / denom
    i1_ref[...] = i1
    i2_ref[...] = i2
    p1_ref[...] = p1
    p2_ref[...] = p2
    psum_ref[...] = jnp.sum(probs, axis=0)


def _ffn_body(be_ref, act_ref, x_ref, w1_ref, b1_ref, w2_ref, b2_ref, y_ref):
    g = pl.program_id(0)

    @pl.when(act_ref[g] > 0)
    def _():
        h = jnp.dot(x_ref[...], w1_ref[0], preferred_element_type=jnp.float32)
        h = jnp.maximum(h + b1_ref[0], 0.0)
        y = jnp.dot(h, w2_ref[0], preferred_element_type=jnp.float32)
        y_ref[...] = y + b2_ref[0]


def kernel(x, gate_w, W1, b1, W2, b2):
    b, s, h = x.shape
    hd = W1.shape[-1]
    x_flat = x.reshape(-1, h)
    n = x_flat.shape[0]

    gw_pad = jnp.zeros((h, 128), jnp.float32).at[:, :E].set(gate_w)
    out_shapes = (
        jax.ShapeDtypeStruct((n, 1), jnp.int32),
        jax.ShapeDtypeStruct((n, 1), jnp.int32),
        jax.ShapeDtypeStruct((n, 1), jnp.float32),
        jax.ShapeDtypeStruct((n, 1), jnp.float32),
        jax.ShapeDtypeStruct((128,), jnp.float32),
    )
    i1b, i2b, p1b, p2b, psum = pl.pallas_call(
        _routing_body,
        out_shape=out_shapes,
    )(x_flat, gw_pad)
    i1, i2 = i1b[:, 0], i2b[:, 0]
    p1, p2 = p1b[:, 0], p2b[:, 0]

    e_all = jnp.concatenate([i1, i2])
    onehot = jax.nn.one_hot(e_all, E, dtype=jnp.int32)
    ranks_all = jnp.cumsum(onehot, axis=0) - onehot
    rank = jnp.sum(ranks_all * onehot, axis=1)
    counts = jnp.sum(onehot, axis=0)
    blocks_per_e = (counts + BLK - 1) // BLK
    cum_blocks = jnp.cumsum(blocks_per_e)
    pad_off = (cum_blocks - blocks_per_e) * BLK
    dst = jnp.sum(pad_off[None, :] * onehot, axis=1) + rank

    G = (K * n) // BLK + E
    R = G * BLK
    total_active = cum_blocks[E - 1]
    g_ids = jnp.arange(G, dtype=jnp.int32)
    g_clamped = jnp.minimum(g_ids, total_active - 1)
    block_expert = jnp.minimum(g_ids // 3, 7)
    active = jnp.ones((G,), jnp.int32)

    tok_all = jnp.concatenate([jnp.arange(n, dtype=jnp.int32)] * K)
    row_tok = jnp.zeros((R,), jnp.int32).at[dst].set(tok_all)

    x_sorted = jnp.take(x_flat, row_tok, axis=0)

    grid_spec = pltpu.PrefetchScalarGridSpec(
        num_scalar_prefetch=2,
        grid=(G,),
        in_specs=[
            pl.BlockSpec((BLK, h), lambda g, be, act: (g, 0)),
            pl.BlockSpec((1, h, hd), lambda g, be, act: (be[g], 0, 0)),
            pl.BlockSpec((1, 1, hd), lambda g, be, act: (be[g], 0, 0)),
            pl.BlockSpec((1, hd, h), lambda g, be, act: (be[g], 0, 0)),
            pl.BlockSpec((1, 1, h), lambda g, be, act: (be[g], 0, 0)),
        ],
        out_specs=pl.BlockSpec((BLK, h), lambda g, be, act: (g, 0)),
    )
    y = pl.pallas_call(
        _ffn_body,
        grid_spec=grid_spec,
        out_shape=jax.ShapeDtypeStruct((R, h), jnp.float32),
        compiler_params=pltpu.CompilerParams(
            dimension_semantics=("arbitrary",),
            vmem_limit_bytes=120 * 1024 * 1024,
        ),
    )(block_expert, active, x_sorted, W1, b1[:, None, :], W2, b2[:, None, :])

    pos1, pos2 = dst[:n], dst[n:]
    out = p1[:, None] * jnp.take(y, pos1, axis=0) + p2[:, None] * jnp.take(y, pos2, axis=0)
    out = out.reshape(b, s, h)

    expert_usage = psum[:E] / n
    log_input = jax.nn.log_softmax(expert_usage, axis=0)
    uniform = jnp.ones_like(expert_usage) / E
    kl = jnp.sum(uniform * (jnp.log(uniform) - log_input)) / E
    load_balance_loss = LB_COEFF * kl
    return out, load_balance_loss


# speedup vs baseline: 1.0944x; 1.0944x over previous
import functools

import jax
import jax.numpy as jnp
from jax.experimental import pallas as pl
from jax.experimental.pallas import tpu as pltpu

E = 8
K = 2
LB_COEFF = 0.01
BLK = 256


def _routing_body(x_ref, gw_ref, i1_ref, i2_ref, p1_ref, p2_ref, psum_ref):
    x = x_ref[...]
    gw = gw_ref[...]
    scores = jnp.dot(x, gw, preferred_element_type=jnp.float32)
    lane = jax.lax.broadcasted_iota(jnp.int32, scores.shape, 1)
    neg_inf = jnp.float32(-jnp.inf)
    scores = jnp.where(lane < E, scores, neg_inf)
    probs = jax.nn.softmax(scores, axis=-1)
    m1 = jnp.max(probs, axis=1, keepdims=True)
    i1 = jnp.min(jnp.where(probs == m1, lane, 128), axis=1, keepdims=True)
    probs2 = jnp.where(lane == i1, jnp.float32(-1.0), probs)
    m2 = jnp.max(probs2, axis=1, keepdims=True)
    i2 = jnp.min(jnp.where(probs2 == m2, lane, 128), axis=1, keepdims=True)
    e2 = jnp.exp(m2 - m1)
    denom = 1.0 + e2
    p1 = 1.0 / denom
    p2 = e2 / denom
    i1_ref[...] = i1
    i2_ref[...] = i2
    p1_ref[...] = p1
    p2_ref[...] = p2
    psum_ref[...] = jnp.sum(probs, axis=0)


def _ffn_body(be_ref, slot_ref, chg_ref, nxt_ref, hasnx_ref, act_ref,
              x_ref, w1_hbm, b1_ref, w2_hbm, b2_ref, y_ref,
              w1b, w2b, sems):
    g = pl.program_id(0)
    slot = slot_ref[g]

    def start_load(e, s):
        pltpu.make_async_copy(w1_hbm.at[e], w1b.at[s], sems.at[s, 0]).start()
        pltpu.make_async_copy(w2_hbm.at[e], w2b.at[s], sems.at[s, 1]).start()

    def wait_load(e, s):
        pltpu.make_async_copy(w1_hbm.at[e], w1b.at[s], sems.at[s, 0]).wait()
        pltpu.make_async_copy(w2_hbm.at[e], w2b.at[s], sems.at[s, 1]).wait()

    @pl.when(g == 0)
    def _():
        start_load(be_ref[0], 0)

    @pl.when(chg_ref[g] == 1)
    def _():
        @pl.when(hasnx_ref[g] == 1)
        def _():
            start_load(nxt_ref[g], 1 - slot)

        wait_load(be_ref[g], slot)

    @pl.when(act_ref[g] > 0)
    def _():
        h1 = jnp.dot(x_ref[...], w1b[slot], preferred_element_type=jnp.float32)
        h1 = jnp.maximum(h1 + b1_ref[0], 0.0)
        y = jnp.dot(h1, w2b[slot], preferred_element_type=jnp.float32)
        y_ref[...] = y + b2_ref[0]


def kernel(x, gate_w, W1, b1, W2, b2):
    b, s, h = x.shape
    hd = W1.shape[-1]
    x_flat = x.reshape(-1, h)
    n = x_flat.shape[0]

    gw_pad = jnp.zeros((h, 128), jnp.float32).at[:, :E].set(gate_w)
    out_shapes = (
        jax.ShapeDtypeStruct((n, 1), jnp.int32),
        jax.ShapeDtypeStruct((n, 1), jnp.int32),
        jax.ShapeDtypeStruct((n, 1), jnp.float32),
        jax.ShapeDtypeStruct((n, 1), jnp.float32),
        jax.ShapeDtypeStruct((128,), jnp.float32),
    )
    i1b, i2b, p1b, p2b, psum = pl.pallas_call(
        _routing_body,
        out_shape=out_shapes,
    )(x_flat, gw_pad)
    i1, i2 = i1b[:, 0], i2b[:, 0]
    p1, p2 = p1b[:, 0], p2b[:, 0]

    e_all = jnp.concatenate([i1, i2])
    onehot = jax.nn.one_hot(e_all, E, dtype=jnp.int32)
    ranks_all = jnp.cumsum(onehot, axis=0) - onehot
    rank = jnp.sum(ranks_all * onehot, axis=1)
    counts = jnp.sum(onehot, axis=0)
    blocks_per_e = (counts + BLK - 1) // BLK
    cum_blocks = jnp.cumsum(blocks_per_e)
    pad_off = (cum_blocks - blocks_per_e) * BLK
    dst = jnp.sum(pad_off[None, :] * onehot, axis=1) + rank

    G = (K * n) // BLK + E
    R = G * BLK
    total_active = cum_blocks[E - 1]
    g_ids = jnp.arange(G, dtype=jnp.int32)
    g_clamped = jnp.minimum(g_ids, total_active - 1)
    block_expert = jnp.searchsorted(cum_blocks, g_clamped, side="right").astype(jnp.int32)
    active = (g_ids < total_active).astype(jnp.int32)

    chg = jnp.concatenate([jnp.ones((1,), jnp.int32),
                           (block_expert[1:] != block_expert[:-1]).astype(jnp.int32)])
    phase = jnp.cumsum(chg) - 1
    slot = (phase % 2).astype(jnp.int32)
    nseg = phase[-1] + 1
    expert_by_seg = jnp.zeros((G,), jnp.int32).at[phase].set(block_expert)
    nxt = expert_by_seg[jnp.minimum(phase + 1, G - 1)].astype(jnp.int32)
    hasnx = (phase + 1 < nseg).astype(jnp.int32)

    tok_all = jnp.concatenate([jnp.arange(n, dtype=jnp.int32)] * K)
    row_tok = jnp.zeros((R,), jnp.int32).at[dst].set(tok_all)

    x_sorted = jnp.take(x_flat, row_tok, axis=0)

    grid_spec = pltpu.PrefetchScalarGridSpec(
        num_scalar_prefetch=6,
        grid=(G,),
        in_specs=[
            pl.BlockSpec((BLK, h), lambda g, *refs: (g, 0)),
            pl.BlockSpec(memory_space=pltpu.MemorySpace.HBM),
            pl.BlockSpec((1, 1, hd), lambda g, be, *refs: (be[g], 0, 0)),
            pl.BlockSpec(memory_space=pltpu.MemorySpace.HBM),
            pl.BlockSpec((1, 1, h), lambda g, be, *refs: (be[g], 0, 0)),
        ],
        out_specs=pl.BlockSpec((BLK, h), lambda g, *refs: (g, 0)),
        scratch_shapes=[
            pltpu.VMEM((2, h, hd), jnp.float32),
            pltpu.VMEM((2, hd, h), jnp.float32),
            pltpu.SemaphoreType.DMA((2, 2)),
        ],
    )
    y = pl.pallas_call(
        _ffn_body,
        grid_spec=grid_spec,
        out_shape=jax.ShapeDtypeStruct((R, h), jnp.float32),
        compiler_params=pltpu.CompilerParams(
            dimension_semantics=("arbitrary",),
            vmem_limit_bytes=120 * 1024 * 1024,
        ),
    )(block_expert, slot, chg, nxt, hasnx, active,
      x_sorted, W1, b1[:, None, :], W2, b2[:, None, :])

    pos1, pos2 = dst[:n], dst[n:]
    out = p1[:, None] * jnp.take(y, pos1, axis=0) + p2[:, None] * jnp.take(y, pos2, axis=0)
    out = out.reshape(b, s, h)

    expert_usage = psum[:E] / n
    log_input = jax.nn.log_softmax(expert_usage, axis=0)
    uniform = jnp.ones_like(expert_usage) / E
    kl = jnp.sum(uniform * (jnp.log(uniform) - log_input)) / E
    load_balance_loss = LB_COEFF * kl
    return out, load_balance_loss


# speedup vs baseline: 1.4407x; 1.3164x over previous
import jax
import jax.numpy as jnp
from jax import lax
from jax.experimental import pallas as pl
from jax.experimental.pallas import tpu as pltpu
from jax.experimental.pallas import tpu_sc as plsc

SC_CORES = 2
SC_SUBCORES = 16
SC_WORKERS = SC_CORES * SC_SUBCORES
PAIRS_PER_W = 128

E = 8
K = 2
LB_COEFF = 0.01
BLK = 256


def _dispatch_body(e_all_hbm, off_hbm, x_hbm, xs_hbm, pos_hbm,
                   ids_v, idx_v, dst_v, pos_v, off_v, rows_v, sem):
    c = lax.axis_index("c")
    s = lax.axis_index("s")
    wid = s * SC_CORES + c
    pair_base = wid * PAIRS_PER_W
    half = SC_SUBCORES * PAIRS_PER_W
    tok_base = jnp.where(pair_base < half, pair_base, pair_base - half)
    lane = lax.iota(jnp.int32, 16)
    pltpu.sync_copy(e_all_hbm.at[pl.ds(pair_base, PAIRS_PER_W)], ids_v)
    pltpu.sync_copy(off_hbm.at[wid], off_v)

    def chunk(ci, carry):
        ids = ids_v[pl.ds(ci * 16, 16)]
        base_vec = off_v[...]
        dst = jnp.zeros((16,), jnp.int32)
        for e in range(E):
            m = ids == e
            mi = m.astype(jnp.int32)
            excl = jnp.cumsum(mi) - mi
            base_e = jnp.sum(jnp.where(lane == e, base_vec, 0))
            dst = jnp.where(m, base_e + excl, dst)
            cnt_e = jnp.sum(mi)
            base_vec = base_vec + jnp.where(lane == e, cnt_e, 0)
        off_v[...] = base_vec
        dst_v[...] = dst
        pos_v[pl.ds(ci * 16, 16)] = dst
        idx_v[...] = tok_base + ci * 16 + lane
        pltpu.async_copy(x_hbm.at[idx_v], rows_v, sem).wait()
        pltpu.async_copy(rows_v, xs_hbm.at[dst_v], sem).wait()
        return carry

    lax.fori_loop(0, PAIRS_PER_W // 16, chunk, 0)
    pltpu.sync_copy(pos_v, pos_hbm.at[pl.ds(pair_base, PAIRS_PER_W)])


def _routing_body(x_ref, gw_ref, i1_ref, i2_ref, p1_ref, p2_ref, psum_ref,
                  c1_ref, c2_ref):
    x = x_ref[...]
    gw = gw_ref[...]
    scores = jnp.dot(x, gw, preferred_element_type=jnp.float32)
    lane = jax.lax.broadcasted_iota(jnp.int32, scores.shape, 1)
    neg_inf = jnp.float32(-jnp.inf)
    scores = jnp.where(lane < E, scores, neg_inf)
    probs = jax.nn.softmax(scores, axis=-1)
    m1 = jnp.max(probs, axis=1, keepdims=True)
    i1 = jnp.min(jnp.where(probs == m1, lane, 128), axis=1, keepdims=True)
    probs2 = jnp.where(lane == i1, jnp.float32(-1.0), probs)
    m2 = jnp.max(probs2, axis=1, keepdims=True)
    i2 = jnp.min(jnp.where(probs2 == m2, lane, 128), axis=1, keepdims=True)
    e2 = jnp.exp(m2 - m1)
    denom = 1.0 + e2
    p1 = 1.0 / denom
    p2 = e2 / denom
    i1_ref[...] = i1
    i2_ref[...] = i2
    p1_ref[...] = p1
    p2_ref[...] = p2
    psum_ref[...] = jnp.sum(probs, axis=0)
    n_tok = x.shape[0]
    sel = (jax.lax.broadcasted_iota(jnp.int32, (16, n_tok), 1) // PAIRS_PER_W
           == jax.lax.broadcasted_iota(jnp.int32, (16, n_tok), 0)).astype(jnp.float32)
    oh1 = (lane == i1).astype(jnp.float32)
    oh2 = (lane == i2).astype(jnp.float32)
    c1_ref[...] = jnp.dot(sel, oh1, preferred_element_type=jnp.float32)
    c2_ref[...] = jnp.dot(sel, oh2, preferred_element_type=jnp.float32)


def _ffn_body(be_ref, slot_ref, chg_ref, nxt_ref, hasnx_ref, act_ref,
              x_ref, w1_hbm, b1_ref, w2_hbm, b2_ref, y_ref,
              w1b, w2b, sems):
    g = pl.program_id(0)
    slot = slot_ref[g]

    def start_load(e, s):
        pltpu.make_async_copy(w1_hbm.at[e], w1b.at[s], sems.at[s, 0]).start()
        pltpu.make_async_copy(w2_hbm.at[e], w2b.at[s], sems.at[s, 1]).start()

    def wait_load(e, s):
        pltpu.make_async_copy(w1_hbm.at[e], w1b.at[s], sems.at[s, 0]).wait()
        pltpu.make_async_copy(w2_hbm.at[e], w2b.at[s], sems.at[s, 1]).wait()

    @pl.when(g == 0)
    def _():
        start_load(be_ref[0], 0)

    @pl.when(chg_ref[g] == 1)
    def _():
        @pl.when(hasnx_ref[g] == 1)
        def _():
            start_load(nxt_ref[g], 1 - slot)

        wait_load(be_ref[g], slot)

    @pl.when(act_ref[g] > 0)
    def _():
        h1 = jnp.dot(x_ref[...], w1b[slot], preferred_element_type=jnp.float32)
        h1 = jnp.maximum(h1 + b1_ref[0], 0.0)
        y = jnp.dot(h1, w2b[slot], preferred_element_type=jnp.float32)
        y_ref[...] = y + b2_ref[0]


def kernel(x, gate_w, W1, b1, W2, b2):
    b, s, h = x.shape
    hd = W1.shape[-1]
    x_flat = x.reshape(-1, h)
    n = x_flat.shape[0]

    gw_pad = jnp.zeros((h, 128), jnp.float32).at[:, :E].set(gate_w)
    out_shapes = (
        jax.ShapeDtypeStruct((n, 1), jnp.int32),
        jax.ShapeDtypeStruct((n, 1), jnp.int32),
        jax.ShapeDtypeStruct((n, 1), jnp.float32),
        jax.ShapeDtypeStruct((n, 1), jnp.float32),
        jax.ShapeDtypeStruct((128,), jnp.float32),
        jax.ShapeDtypeStruct((16, 128), jnp.float32),
        jax.ShapeDtypeStruct((16, 128), jnp.float32),
    )
    i1b, i2b, p1b, p2b, psum, c1b, c2b = pl.pallas_call(
        _routing_body,
        out_shape=out_shapes,
    )(x_flat, gw_pad)
    p1, p2 = p1b[:, 0], p2b[:, 0]

    cnt_chunk = jnp.concatenate([c1b[:, :E], c2b[:, :E]]).astype(jnp.int32)
    counts = jnp.sum(cnt_chunk, axis=0)
    blocks_per_e = (counts + BLK - 1) // BLK
    cum_blocks = jnp.cumsum(blocks_per_e)
    pad_off = (cum_blocks - blocks_per_e) * BLK
    off = pad_off[None, :] + jnp.cumsum(cnt_chunk, axis=0) - cnt_chunk
    off_pad = jnp.zeros((SC_WORKERS, 16), jnp.int32).at[:, :E].set(off)

    G = (K * n) // BLK + E
    R = G * BLK
    total_active = cum_blocks[E - 1]
    g_ids = jnp.arange(G, dtype=jnp.int32)
    g_clamped = jnp.minimum(g_ids, total_active - 1)
    block_expert = jnp.searchsorted(cum_blocks, g_clamped, side="right").astype(jnp.int32)
    active = (g_ids < total_active).astype(jnp.int32)

    chg = jnp.concatenate([jnp.ones((1,), jnp.int32),
                           (block_expert[1:] != block_expert[:-1]).astype(jnp.int32)])
    phase = jnp.cumsum(chg) - 1
    slot = (phase % 2).astype(jnp.int32)
    nseg = phase[-1] + 1
    expert_by_seg = jnp.zeros((G,), jnp.int32).at[phase].set(block_expert)
    nxt = expert_by_seg[jnp.minimum(phase + 1, G - 1)].astype(jnp.int32)
    hasnx = (phase + 1 < nseg).astype(jnp.int32)

    e_all = jnp.concatenate([i1b[:, 0], i2b[:, 0]])
    mesh = plsc.VectorSubcoreMesh(core_axis_name="c", subcore_axis_name="s")
    dispatch = pl.kernel(
        _dispatch_body,
        out_type=(jax.ShapeDtypeStruct((R, h), jnp.float32),
                  jax.ShapeDtypeStruct((K * n,), jnp.int32)),
        mesh=mesh,
        compiler_params=pltpu.CompilerParams(needs_layout_passes=False),
        scratch_types=[
            pltpu.VMEM((PAIRS_PER_W,), jnp.int32),
            pltpu.VMEM((16,), jnp.int32),
            pltpu.VMEM((16,), jnp.int32),
            pltpu.VMEM((PAIRS_PER_W,), jnp.int32),
            pltpu.VMEM((16,), jnp.int32),
            pltpu.VMEM((16, h), jnp.float32),
            pltpu.SemaphoreType.DMA,
        ],
    )
    x_sorted, pos = dispatch(e_all, off_pad, x_flat)
    dst = pos

    grid_spec = pltpu.PrefetchScalarGridSpec(
        num_scalar_prefetch=6,
        grid=(G,),
        in_specs=[
            pl.BlockSpec((BLK, h), lambda g, *refs: (g, 0)),
            pl.BlockSpec(memory_space=pltpu.MemorySpace.HBM),
            pl.BlockSpec((1, 1, hd), lambda g, be, *refs: (be[g], 0, 0)),
            pl.BlockSpec(memory_space=pltpu.MemorySpace.HBM),
            pl.BlockSpec((1, 1, h), lambda g, be, *refs: (be[g], 0, 0)),
        ],
        out_specs=pl.BlockSpec((BLK, h), lambda g, *refs: (g, 0)),
        scratch_shapes=[
            pltpu.VMEM((2, h, hd), jnp.float32),
            pltpu.VMEM((2, hd, h), jnp.float32),
            pltpu.SemaphoreType.DMA((2, 2)),
        ],
    )
    y = pl.pallas_call(
        _ffn_body,
        grid_spec=grid_spec,
        out_shape=jax.ShapeDtypeStruct((R, h), jnp.float32),
        compiler_params=pltpu.CompilerParams(
            dimension_semantics=("arbitrary",),
            vmem_limit_bytes=120 * 1024 * 1024,
        ),
    )(block_expert, slot, chg, nxt, hasnx, active,
      x_sorted, W1, b1[:, None, :], W2, b2[:, None, :])

    pos1, pos2 = dst[:n], dst[n:]
    out = p1[:, None] * jnp.take(y, pos1, axis=0) + p2[:, None] * jnp.take(y, pos2, axis=0)
    out = out.reshape(b, s, h)

    expert_usage = psum[:E] / n
    log_input = jax.nn.log_softmax(expert_usage, axis=0)
    uniform = jnp.ones_like(expert_usage) / E
    kl = jnp.sum(uniform * (jnp.log(uniform) - log_input)) / E
    load_balance_loss = LB_COEFF * kl
    return out, load_balance_loss
